# baseline (speedup 1.0000x reference)
import functools
import math

import jax
import jax.numpy as jnp
from jax import lax
from jax.experimental import pallas as pl
from jax.experimental.pallas import tpu as pltpu
from jax.experimental.pallas import tpu_sc as plsc

N_ROWS = 16384
SIZE = 1000
EPS = 0.1 / (SIZE - 2)
CONF = 0.9
C_ENT = CONF * math.log(CONF) + (SIZE - 2) * EPS * math.log(EPS)
CME = CONF - EPS

R_SC = 9216

NC = 2
NS = 16
L = 16
NW = NC * NS
SPW = R_SC // NW
RB = 32
NG = SPW // RB
NCH = SIZE // L
FULL = NCH * L
NACC = 8
UNROLL = 8
NBUF = 3

KSTREAM = 2
TBS = 512
TBT = KSTREAM * TBS
TG = (N_ROWS - R_SC) // TBT
TOFF = R_SC // TBS


def _sc_loss_body(x_hbm, tgt_hbm, out_hbm, t_v, buf0, buf1, buf2, out_v,
                  sem0, sem1, sem2):
    wid = lax.axis_index("s") * NC + lax.axis_index("c")
    base = wid * SPW
    pltpu.sync_copy(tgt_hbm.at[pl.ds(base, SPW)], t_v)
    iota = lax.iota(jnp.int32, L)
    zeros = jnp.zeros((L,), jnp.int32)
    zf = jnp.zeros((L,), jnp.float32)
    tail_mask = iota >= (L - (SIZE - FULL))

    bufs = (buf0, buf1, buf2)
    sems = (sem0, sem1, sem2)

    def start_group(g):
        return pltpu.async_copy(
            x_hbm.at[pl.ds(base + g * RB, RB)],
            bufs[g % NBUF], sems[g % NBUF])

    cps = [None] * NBUF
    for g0 in range(min(NBUF - 1, NG)):
        cps[g0 % NBUF] = start_group(g0)

    acc_all = zf
    acc_pad = zf
    acc_b = zf
    acc_g = zf
    acc_n = zeros

    for g in range(NG):
        if g + NBUF - 1 < NG:
            cps[(g + NBUF - 1) % NBUF] = start_group(g + NBUF - 1)
        cps[g % NBUF].wait()
        buf = bufs[g % NBUF]

        def chunk_body(i, accs, buf=buf):
            accs = list(accs)
            for r in range(RB):
                accs[r % NACC] = accs[r % NACC] + buf[r, pl.ds(i * L, L)]
            return tuple(accs)

        parts = lax.fori_loop(0, NCH, chunk_body, (zf,) * NACC)
        bulk = functools.reduce(lambda a, b: a + b, parts)
        for r in range(RB):
            bulk = bulk + jnp.where(tail_mask, buf[r, pl.ds(FULL - 8, L)], zf)
        acc_all = acc_all + bulk

        for h in range(RB // L):
            rowv = zeros + h * L + iota
            tv = t_v[pl.ds(g * RB + h * L, L)]
            nonpad = tv != 0
            x0v = plsc.load_gather(buf, [rowv, zeros])
            xtv = plsc.load_gather(buf, [rowv, tv])
            acc_b = acc_b + jnp.where(nonpad, x0v, zf)
            acc_g = acc_g + jnp.where(nonpad, xtv, zf)
            acc_n = acc_n + jnp.where(nonpad, zeros + 1, zeros)

            padmask = tv == 0
            npad = lax.reduce_max(
                plsc.all_reduce_population_count(padmask), (0,))
            ntrips = jnp.where(npad > 0, SIZE // UNROLL, 0)

            def pad_body(i, acc, buf=buf, rowv=rowv, padmask=padmask):
                for u in range(UNROLL):
                    cv = zeros + (i * UNROLL + u)
                    v = plsc.load_gather(buf, [rowv, cv])
                    acc = acc + jnp.where(padmask, v, zf)
                return acc

            acc_pad = acc_pad + lax.fori_loop(0, ntrips, pad_body, zf)

    out_v[0, pl.ds(0, L)] = acc_all
    out_v[1, pl.ds(0, L)] = acc_pad
    out_v[2, pl.ds(0, L)] = acc_b
    out_v[3, pl.ds(0, L)] = acc_g
    out_v[4, pl.ds(0, L)] = acc_n.astype(jnp.float32)
    pltpu.sync_copy(out_v, out_hbm.at[wid])


@functools.lru_cache(maxsize=None)
def _make_sc_loss():
    return functools.partial(
        pl.kernel,
        mesh=plsc.VectorSubcoreMesh(core_axis_name="c", subcore_axis_name="s"),
        out_type=jax.ShapeDtypeStruct((NW, 5, L), jnp.float32),
        compiler_params=pltpu.CompilerParams(
            needs_layout_passes=False, skip_device_barrier=True),
        scratch_types=[
            pltpu.VMEM((SPW,), jnp.int32),
            pltpu.VMEM((RB, SIZE), jnp.float32),
            pltpu.VMEM((RB, SIZE), jnp.float32),
            pltpu.VMEM((RB, SIZE), jnp.float32),
            pltpu.VMEM((5, L), jnp.float32),
            pltpu.SemaphoreType.DMA,
            pltpu.SemaphoreType.DMA,
            pltpu.SemaphoreType.DMA,
        ],
    )(_sc_loss_body)


def _tc_body(*refs):
    x_refs = refs[:KSTREAM]
    t_refs = refs[KSTREAM:2 * KSTREAM]
    o_ref = refs[2 * KSTREAM]
    i = pl.program_id(0)
    s = jnp.float32(0.0)
    for k, x_ref in enumerate(x_refs):
        xb = x_ref[...]
        tk = t_refs[k][0]
        nonpad = tk != 0
        rs = jnp.sum(xb, axis=1, keepdims=True)
        x0 = xb[:, 0:1]
        cols = jax.lax.broadcasted_iota(jnp.int32, (TBS, SIZE), 1)
        gk = jnp.sum(
            jnp.where(cols == tk, xb, jnp.float32(0.0)),
            axis=1, keepdims=True,
        )
        contrib = jnp.where(
            nonpad,
            jnp.float32(C_ENT)
            - jnp.float32(EPS) * (rs - x0)
            - jnp.float32(CME) * gk,
            jnp.float32(0.0),
        )
        s = s + jnp.sum(contrib)

    @pl.when(i == 0)
    def _init():
        o_ref[0, 0] = jnp.float32(0.0)

    o_ref[0, 0] += s


def _tc_loss(x, t32):
    t3 = t32.reshape(N_ROWS // TBS, TBS, 1)
    x_specs = [
        pl.BlockSpec((TBS, SIZE), lambda i, k=k: (TOFF + i * KSTREAM + k, 0))
        for k in range(KSTREAM)
    ]
    t_specs = [
        pl.BlockSpec(
            (1, TBS, 1), lambda i, k=k: (TOFF + i * KSTREAM + k, 0, 0))
        for k in range(KSTREAM)
    ]
    out = pl.pallas_call(
        _tc_body,
        grid=(TG,),
        in_specs=x_specs + t_specs,
        out_specs=pl.BlockSpec(
            (1, 1), lambda i: (0, 0), memory_space=pltpu.SMEM
        ),
        out_shape=jax.ShapeDtypeStruct((1, 1), jnp.float32),
        compiler_params=pltpu.CompilerParams(
            dimension_semantics=("arbitrary",),
            skip_device_barrier=True,
        ),
    )(*([x] * KSTREAM), *([t3] * KSTREAM))
    return out[0, 0]


def kernel(x, target):
    t32 = target.astype(jnp.int32)
    parts = _make_sc_loss()(x[:R_SC], t32)
    tc_part = _tc_loss(x, t32) if TG > 0 else jnp.float32(0.0)
    s_all = jnp.sum(parts[:, 0, :])
    s_pad = jnp.sum(parts[:, 1, :])
    s_x0 = jnp.sum(parts[:, 2, :])
    s_g = jnp.sum(parts[:, 3, :])
    cnt = jnp.sum(parts[:, 4, :])
    return (tc_part
            + cnt * jnp.float32(C_ENT)
            - jnp.float32(EPS) * (s_all - s_pad - s_x0)
            - jnp.float32(CME) * s_g)


# speedup vs baseline: 1.4971x; 1.4971x over previous
import jax
import jax.numpy as jnp
from jax.experimental import pallas as pl
from jax.experimental.pallas import tpu as pltpu
import math

N_ROWS = 16384
SIZE = 1000
EPS = 0.1 / (SIZE - 2)
CONF = 0.9
C_ENT = CONF * math.log(CONF) + (SIZE - 2) * EPS * math.log(EPS)
CME = CONF - EPS

CH = 256
NCHUNK = N_ROWS // CH
NBUFT = 8
NSUP = NCHUNK // NBUFT


def _body(x_hbm, t_ref, o_ref, *scratch):
    bufs = scratch[:NBUFT]
    sems = scratch[NBUFT:2 * NBUFT]

    def start(idx):
        pass

    cps = []
    for j in range(NBUFT - 1):
        pltpu.make_async_copy(
            x_hbm.at[pl.ds(j * CH, CH), :], bufs[j], sems[j]).start()

    def chunk_compute(idx, buf):
        xb = buf[...]
        tk = t_ref[pl.ds(idx * CH, CH), :]
        nonpad = tk != 0
        rs = jnp.sum(xb, axis=1, keepdims=True)
        x0 = xb[:, 0:1]
        cols = jax.lax.broadcasted_iota(jnp.int32, (CH, SIZE), 1)
        gk = jnp.sum(
            jnp.where(cols == tk, xb, jnp.float32(0.0)),
            axis=1, keepdims=True,
        )
        contrib = jnp.where(
            nonpad,
            jnp.float32(C_ENT)
            - jnp.float32(EPS) * (rs - x0)
            - jnp.float32(CME) * gk,
            jnp.float32(0.0),
        )
        return jnp.sum(contrib)

    def sup_body(s, acc):
        for j in range(NBUFT):
            idx = s * NBUFT + j
            pltpu.make_async_copy(
                x_hbm.at[pl.ds(idx * CH, CH), :], bufs[j], sems[j]).wait()
            nxt = idx + NBUFT - 1

            @pl.when(nxt < NCHUNK)
            def _():
                pltpu.make_async_copy(
                    x_hbm.at[pl.ds(nxt * CH, CH), :],
                    bufs[(j + NBUFT - 1) % NBUFT],
                    sems[(j + NBUFT - 1) % NBUFT]).start()

            acc = acc + chunk_compute(idx, bufs[j])
        return acc

    total = jax.lax.fori_loop(0, NSUP, sup_body, jnp.float32(0.0))
    o_ref[0, 0] = total


def kernel(x, target):
    t32 = target.astype(jnp.int32).reshape(N_ROWS, 1)
    out = pl.pallas_call(
        _body,
        in_specs=[
            pl.BlockSpec(memory_space=pl.ANY),
            pl.BlockSpec(memory_space=pltpu.VMEM),
        ],
        out_specs=pl.BlockSpec(memory_space=pltpu.SMEM),
        out_shape=jax.ShapeDtypeStruct((1, 1), jnp.float32),
        scratch_shapes=(
            [pltpu.VMEM((CH, SIZE), jnp.float32) for _ in range(NBUFT)]
            + [pltpu.SemaphoreType.DMA for _ in range(NBUFT)]
        ),
    )(x, t32)
    return out[0, 0]
